# baseline (speedup 1.0000x reference)
import functools

import jax
import jax.numpy as jnp
from jax import lax
from jax.experimental import pallas as pl
from jax.experimental.pallas import tpu as pltpu
from jax.experimental.pallas import tpu_sc as plsc

BATCH = 16384
D = 32
NEG = 4
NW = 32
NUSER = 1000000
RANGE = 31232
CW = 512
NCH = 62
SWEEP_END = 999936
WLCAP = 8192
HLCAP = 2048
STG_ROWS = 144
U_DUMMY = BATCH
I_DUMMY = 5 * BATCH


def _iota16():
    return lax.iota(jnp.int32, 16)


def _splat(v):
    return jnp.full((16,), v, jnp.int32)


def _scan_src(src_hbm, nbatches, slot_off, lo_v, hi_v, idxb, wli, wls, wcnt):
    lane = _iota16()

    def batch_body(bi, wc):
        pltpu.sync_copy(src_hbm.at[pl.ds(bi * 4096, 4096)], idxb)

        def group_body(k, wc2):
            wi = idxb[pl.ds(k * 16, 16)]
            m = (wi >= lo_v) & (wi < hi_v)
            m32 = m.astype(jnp.int32)
            pos = jnp.minimum(wc2 + plsc.cumsum(m32) - 1, _splat(WLCAP - 1))
            plsc.store_scatter(wli, [pos], wi, mask=m)
            slot = _splat(slot_off) + bi * 4096 + k * 16 + lane
            plsc.store_scatter(wls, [pos], slot, mask=m)
            return wc2 + plsc.all_reduce_population_count(m)

        return lax.fori_loop(0, 256, group_body, wc)

    return lax.fori_loop(0, nbatches, batch_body, wcnt)


def _bpr_extract_body(tu, ti, user_h, pos_h, negf_h, tailu_h, taili_h,
                      ubuf, ibuf,
                      idxb, wli, wls, hlu, hls, stg, sl2, chk0, tailv,
                      semc, semf):
    c_ax = lax.axis_index("c")
    s_ax = lax.axis_index("s")
    wid = s_ax * 2 + c_ax
    lane = _iota16()
    lo = wid * RANGE
    lo_v = _splat(0) + lo

    def dummy_slots(row, dummy):
        for g in range(8):
            plsc.store_scatter(sl2, [_splat(row), g * 16 + lane],
                               _splat(dummy), mask=None)

    def fire(buf_hbm):
        cp = pltpu.async_copy(stg.at[pl.ds(0, 128)], buf_hbm.at[sl2.at[0]],
                              semf)
        cp.wait()

    def stage_b(chkb, tiled_chunk, hcnt_vec, scnt_vec, buf_hbm, dummy):
        hn = jnp.max(hcnt_vec)

        def g_body(g, scnt):
            ul = hlu[pl.ds(g * 16, 16)]
            sl = hls[pl.ds(g * 16, 16)]
            m = (g * 16 + lane) < hcnt_vec
            p = scnt + plsc.cumsum(m.astype(jnp.int32)) - 1
            p = jnp.minimum(p, _splat(STG_ROWS - 1))
            if tiled_chunk:
                q = jnp.right_shift(ul, 7)
                l = jnp.bitwise_and(ul, _splat(127))
            for d in range(D):
                dv = _splat(d)
                if tiled_chunk:
                    v = plsc.load_gather(
                        chkb, [q, _splat(d >> 3), _splat(d & 7), l], mask=m)
                else:
                    v = plsc.load_gather(chkb, [ul, dv], mask=m)
                plsc.store_scatter(stg, [p, dv], v, mask=m)
            plsc.store_scatter(sl2, [jnp.right_shift(p, 7),
                                     jnp.bitwise_and(p, _splat(127))],
                               sl, mask=m)
            scnt2 = scnt + plsc.all_reduce_population_count(m)
            fired = jnp.max(scnt2) >= 128

            @pl.when(fired)
            def _():
                fire(buf_hbm)
                for r in range(16):
                    for q in range(8):
                        stg[r, pl.ds(q * 16, 16)] = (
                            stg[128 + r, pl.ds(q * 16, 16)])
                for q in range(8):
                    sl2[0, pl.ds(q * 16, 16)] = sl2[1, pl.ds(q * 16, 16)]
                dummy_slots(1, dummy)

            return jnp.where(fired, scnt2 - 128, scnt2)

        return lax.fori_loop(0, (hn + 15) // 16, g_body, scnt_vec)

    def stage_a(clo_v, cw, wn_vec):
        wn = jnp.max(wn_vec)

        def k_body(k, hcnt):
            wi = wli[pl.ds(k * 16, 16)]
            m = ((wi >= clo_v) & (wi < clo_v + cw)
                 & ((k * 16 + lane) < wn_vec))
            pos = jnp.minimum(hcnt + plsc.cumsum(m.astype(jnp.int32)) - 1,
                              _splat(HLCAP - 1))
            plsc.store_scatter(hlu, [pos], wi - clo_v, mask=m)
            ws = wls[pl.ds(k * 16, 16)]
            plsc.store_scatter(hls, [pos], ws, mask=m)
            return hcnt + plsc.all_reduce_population_count(m)

        return lax.fori_loop(0, (wn + 15) // 16, k_body, _splat(0))

    def run_phase(table_v, tail_hbm, sources, buf_hbm, dummy):
        hi_v = jnp.where(wid == 31, NUSER, lo + RANGE) + _splat(0)
        wcnt = _splat(0)
        for (src, nb, soff) in sources:
            wcnt = _scan_src(src, nb, soff, lo_v, hi_v,
                             idxb, wli, wls, wcnt)
        dummy_slots(0, dummy)
        dummy_slots(1, dummy)
        pltpu.sync_copy(tail_hbm, tailv)

        def chunk_body(c, scnt):
            for q in range(4):
                s = pl.multiple_of(lo + c * CW + q * 128, 128)
                pltpu.sync_copy(table_v.at[:, :, pl.ds(s, 128)], chk0.at[q])
            hcnt = stage_a(_splat(0) + (lo + c * CW), CW, wcnt)
            return stage_b(chk0, True, hcnt, scnt, buf_hbm, dummy)

        scnt = lax.fori_loop(0, NCH, chunk_body, _splat(0))
        hcnt = stage_a(_splat(SWEEP_END), NUSER - SWEEP_END, wcnt)
        scnt = stage_b(tailv, False, hcnt, scnt, buf_hbm, dummy)
        fire(buf_hbm)

    run_phase(tu, tailu_h, [(user_h, 4, 0)], ubuf, U_DUMMY)
    run_phase(ti, taili_h, [(pos_h, 4, 0), (negf_h, 16, BATCH)], ibuf,
              I_DUMMY)


def _bpr_dots_body(ubuf, ibuf, outp, ulines, plines, nlines, outv, sem):
    c_ax = lax.axis_index("c")
    s_ax = lax.axis_index("s")
    wid = s_ax * 2 + c_ax
    lane = _iota16()
    base = wid * 512

    def sb_body(sb, carry):
        sbase = base + sb * 128
        pltpu.sync_copy(ubuf.at[pl.ds(sbase, 128)], ulines)
        pltpu.sync_copy(ibuf.at[pl.ds(sbase, 128)], plines)
        pltpu.sync_copy(ibuf.at[pl.ds(BATCH + sbase * 4, 512)], nlines)

        def g_body(g, carry2):
            lb = g * 16 + lane
            accp = jnp.zeros((16,), jnp.float32)
            accn = [jnp.zeros((16,), jnp.float32) for _ in range(NEG)]
            for d in range(D):
                dv = _splat(d)
                u = plsc.load_gather(ulines, [lb, dv])
                p = plsc.load_gather(plines, [lb, dv])
                accp = accp + u * p
                for j in range(NEG):
                    n = plsc.load_gather(nlines, [lb * NEG + j, dv])
                    accn[j] = accn[j] + u * n
            orow = jnp.right_shift(lb, 4)
            ocol0 = jnp.bitwise_and(lb, _splat(15)) * 8
            for cc in range(NEG):
                plsc.store_scatter(outv, [orow, ocol0 + cc], accp, mask=None)
            for j in range(NEG):
                plsc.store_scatter(outv, [orow, ocol0 + NEG + j], accn[j],
                                   mask=None)
            return carry2

        lax.fori_loop(0, 8, g_body, 0)
        pltpu.sync_copy(outv, outp.at[pl.ds(wid * 32 + sb * 8, 8)])
        return carry

    lax.fori_loop(0, 4, sb_body, 0)


@jax.jit
def _bpr(user, pos, negf, tu, ti, tailu, taili):
    mesh = plsc.VectorSubcoreMesh(core_axis_name="c", subcore_axis_name="s")
    cp = pltpu.CompilerParams(needs_layout_passes=False,
                              use_tc_tiling_on_sc=True)
    k1 = pl.kernel(
        _bpr_extract_body,
        out_type=(jax.ShapeDtypeStruct((BATCH + 1, 128), jnp.float32),
                  jax.ShapeDtypeStruct((5 * BATCH + 1, 128), jnp.float32)),
        mesh=mesh,
        scratch_types=[
            pltpu.VMEM((4096,), jnp.int32),
            pltpu.VMEM((WLCAP,), jnp.int32),
            pltpu.VMEM((WLCAP,), jnp.int32),
            pltpu.VMEM((HLCAP,), jnp.int32),
            pltpu.VMEM((HLCAP,), jnp.int32),
            pltpu.VMEM((STG_ROWS, 128), jnp.float32),
            pltpu.VMEM((2, 128), jnp.int32),
            pltpu.VMEM((4, 4, 8, 128), jnp.float32),
            pltpu.VMEM((64, 128), jnp.float32),
            pltpu.SemaphoreType.DMA,
            pltpu.SemaphoreType.DMA,
        ],
        compiler_params=cp,
    )
    ubuf, ibuf = k1(tu, ti, user, pos, negf, tailu, taili)
    k2 = pl.kernel(
        _bpr_dots_body,
        out_type=jax.ShapeDtypeStruct((1024, 128), jnp.float32),
        mesh=mesh,
        scratch_types=[
            pltpu.VMEM((128, 128), jnp.float32),
            pltpu.VMEM((128, 128), jnp.float32),
            pltpu.VMEM((512, 128), jnp.float32),
            pltpu.VMEM((8, 128), jnp.float32),
            pltpu.SemaphoreType.DMA,
        ],
        compiler_params=cp,
    )
    outp = k2(ubuf, ibuf)
    return outp.reshape(BATCH, 2 * NEG)


def kernel(user, pos_item, neg_item, user_table, item_table):
    user = user.astype(jnp.int32)
    pos = pos_item.astype(jnp.int32)
    negf = neg_item.astype(jnp.int32).reshape(BATCH * NEG)
    tu = user_table.T.reshape(4, 8, NUSER)
    ti = item_table.T.reshape(4, 8, NUSER)
    tailu = jnp.pad(user_table[SWEEP_END:], ((0, 0), (0, 128 - D)))
    taili = jnp.pad(item_table[SWEEP_END:], ((0, 0), (0, 128 - D)))
    return _bpr(user, pos, negf, tu, ti, tailu, taili)


# speedup vs baseline: 1.5292x; 1.5292x over previous
import functools

import jax
import jax.numpy as jnp
from jax import lax
from jax.experimental import pallas as pl
from jax.experimental.pallas import tpu as pltpu
from jax.experimental.pallas import tpu_sc as plsc

BATCH = 16384
D = 32
NEG = 4
NW = 32
NUSER = 1000000
RANGE = 31232
CW = 512
NCH = 62
SWEEP_END = 999936
WLCAP = 8192
HLCAP = 2048
STG_ROWS = 144
U_DUMMY = BATCH
I_DUMMY = 5 * BATCH


def _iota16():
    return lax.iota(jnp.int32, 16)


def _splat(v):
    return jnp.full((16,), v, jnp.int32)


def _scan_src(src_hbm, nbatches, slot_off, lo_v, hi_v, idxb, wli, wls, wcnt):
    lane = _iota16()

    def batch_body(bi, wc):
        pltpu.sync_copy(src_hbm.at[pl.ds(bi * 4096, 4096)], idxb)

        def group_body(k, wc2):
            wi = idxb[pl.ds(k * 16, 16)]
            m = (wi >= lo_v) & (wi < hi_v)
            m32 = m.astype(jnp.int32)
            pos = jnp.minimum(wc2 + plsc.cumsum(m32) - 1, _splat(WLCAP - 1))
            plsc.store_scatter(wli, [pos], wi, mask=m)
            slot = _splat(slot_off) + bi * 4096 + k * 16 + lane
            plsc.store_scatter(wls, [pos], slot, mask=m)
            return wc2 + plsc.all_reduce_population_count(m)

        return lax.fori_loop(0, 256, group_body, wc)

    return lax.fori_loop(0, nbatches, batch_body, wcnt)


def _bpr_extract_body(tu, ti, user_h, pos_h, negf_h, tailu_h, taili_h,
                      ubuf, ibuf,
                      idxb, wli, wls, hlu, hls, stg, sl2, chk0, chk1, tailv,
                      semc, semf):
    c_ax = lax.axis_index("c")
    s_ax = lax.axis_index("s")
    wid = s_ax * 2 + c_ax
    lane = _iota16()
    lo = wid * RANGE
    lo_v = _splat(0) + lo

    def dummy_slots(row, dummy):
        for g in range(8):
            plsc.store_scatter(sl2, [_splat(row), g * 16 + lane],
                               _splat(dummy), mask=None)

    def fire(buf_hbm):
        cp = pltpu.async_copy(stg.at[pl.ds(0, 128)], buf_hbm.at[sl2.at[0]],
                              semf)
        cp.wait()

    def stage_b(chkb, tiled_chunk, hcnt_vec, scnt_vec, buf_hbm, dummy):
        hn = jnp.max(hcnt_vec)

        def g_body(g, scnt):
            ul = hlu[pl.ds(g * 16, 16)]
            sl = hls[pl.ds(g * 16, 16)]
            m = (g * 16 + lane) < hcnt_vec
            p = scnt + plsc.cumsum(m.astype(jnp.int32)) - 1
            p = jnp.minimum(p, _splat(STG_ROWS - 1))
            for d in range(D):
                dv = _splat(d)
                if tiled_chunk:
                    v = plsc.load_gather(
                        chkb, [_splat(d >> 3), _splat(d & 7), ul], mask=m)
                else:
                    v = plsc.load_gather(chkb, [ul, dv], mask=m)
                plsc.store_scatter(stg, [p, dv], v, mask=m)
            plsc.store_scatter(sl2, [jnp.right_shift(p, 7),
                                     jnp.bitwise_and(p, _splat(127))],
                               sl, mask=m)
            scnt2 = scnt + plsc.all_reduce_population_count(m)
            fired = jnp.max(scnt2) >= 128

            @pl.when(fired)
            def _():
                fire(buf_hbm)
                for r in range(16):
                    for q in range(8):
                        stg[r, pl.ds(q * 16, 16)] = (
                            stg[128 + r, pl.ds(q * 16, 16)])
                for q in range(8):
                    sl2[0, pl.ds(q * 16, 16)] = sl2[1, pl.ds(q * 16, 16)]
                dummy_slots(1, dummy)

            return jnp.where(fired, scnt2 - 128, scnt2)

        return lax.fori_loop(0, (hn + 15) // 16, g_body, scnt_vec)

    def stage_a(clo_v, cw, wn_vec):
        wn = jnp.max(wn_vec)

        def k_body(k, hcnt):
            wi = wli[pl.ds(k * 16, 16)]
            m = ((wi >= clo_v) & (wi < clo_v + cw)
                 & ((k * 16 + lane) < wn_vec))
            pos = jnp.minimum(hcnt + plsc.cumsum(m.astype(jnp.int32)) - 1,
                              _splat(HLCAP - 1))
            plsc.store_scatter(hlu, [pos], wi - clo_v, mask=m)
            ws = wls[pl.ds(k * 16, 16)]
            plsc.store_scatter(hls, [pos], ws, mask=m)
            return hcnt + plsc.all_reduce_population_count(m)

        return lax.fori_loop(0, (wn + 15) // 16, k_body, _splat(0))

    def run_phase(table_v, tail_hbm, sources, buf_hbm, dummy):
        hi_v = jnp.where(wid == 31, NUSER, lo + RANGE) + _splat(0)
        wcnt = _splat(0)
        for (src, nb, soff) in sources:
            wcnt = _scan_src(src, nb, soff, lo_v, hi_v,
                             idxb, wli, wls, wcnt)
        dummy_slots(0, dummy)
        dummy_slots(1, dummy)
        pltpu.sync_copy(tail_hbm, tailv)

        def start_chunk(c, chkb):
            s = pl.multiple_of(lo + c * CW, 128)
            pltpu.async_copy(table_v.at[:, :, pl.ds(s, CW)], chkb, semc)

        def drain_chunk(c, chkb):
            s = pl.multiple_of(lo + c * CW, 128)
            pltpu.make_async_copy(table_v.at[:, :, pl.ds(s, CW)],
                                  chkb, semc).wait()

        start_chunk(0, chk0)
        start_chunk(1, chk1)

        def pair_body(c2, scnt):
            for b, chkb in ((0, chk0), (1, chk1)):
                c = 2 * c2 + b
                drain_chunk(c, chkb)
                hcnt = stage_a(_splat(0) + (lo + c * CW), CW, wcnt)
                scnt = stage_b(chkb, True, hcnt, scnt, buf_hbm, dummy)

                @pl.when(c2 < 30)
                def _():
                    start_chunk(c + 2, chkb)
            return scnt

        scnt = lax.fori_loop(0, NCH // 2, pair_body, _splat(0))
        hcnt = stage_a(_splat(SWEEP_END), NUSER - SWEEP_END, wcnt)
        scnt = stage_b(tailv, False, hcnt, scnt, buf_hbm, dummy)
        fire(buf_hbm)

    run_phase(tu, tailu_h, [(user_h, 4, 0)], ubuf, U_DUMMY)
    run_phase(ti, taili_h, [(pos_h, 4, 0), (negf_h, 16, BATCH)], ibuf,
              I_DUMMY)


def _bpr_dots_body(ubuf, ibuf, outp, ulines, plines, nlines, outv, sem):
    c_ax = lax.axis_index("c")
    s_ax = lax.axis_index("s")
    wid = s_ax * 2 + c_ax
    lane = _iota16()
    base = wid * 512

    def sb_body(sb, carry):
        sbase = base + sb * 128
        pltpu.sync_copy(ubuf.at[pl.ds(sbase, 128)], ulines)
        pltpu.sync_copy(ibuf.at[pl.ds(sbase, 128)], plines)
        pltpu.sync_copy(ibuf.at[pl.ds(BATCH + sbase * 4, 512)], nlines)

        def g_body(g, carry2):
            lb = g * 16 + lane
            accp = jnp.zeros((16,), jnp.float32)
            accn = [jnp.zeros((16,), jnp.float32) for _ in range(NEG)]
            for d in range(D):
                dv = _splat(d)
                u = plsc.load_gather(ulines, [lb, dv])
                p = plsc.load_gather(plines, [lb, dv])
                accp = accp + u * p
                for j in range(NEG):
                    n = plsc.load_gather(nlines, [lb * NEG + j, dv])
                    accn[j] = accn[j] + u * n
            orow = jnp.right_shift(lb, 4)
            ocol0 = jnp.bitwise_and(lb, _splat(15)) * 8
            for cc in range(NEG):
                plsc.store_scatter(outv, [orow, ocol0 + cc], accp, mask=None)
            for j in range(NEG):
                plsc.store_scatter(outv, [orow, ocol0 + NEG + j], accn[j],
                                   mask=None)
            return carry2

        lax.fori_loop(0, 8, g_body, 0)
        pltpu.sync_copy(outv, outp.at[pl.ds(wid * 32 + sb * 8, 8)])
        return carry

    lax.fori_loop(0, 4, sb_body, 0)


@jax.jit
def _bpr(user, pos, negf, tu, ti, tailu, taili):
    mesh = plsc.VectorSubcoreMesh(core_axis_name="c", subcore_axis_name="s")
    cp = pltpu.CompilerParams(needs_layout_passes=False,
                              use_tc_tiling_on_sc=True)
    k1 = pl.kernel(
        _bpr_extract_body,
        out_type=(jax.ShapeDtypeStruct((BATCH + 1, 128), jnp.float32),
                  jax.ShapeDtypeStruct((5 * BATCH + 1, 128), jnp.float32)),
        mesh=mesh,
        scratch_types=[
            pltpu.VMEM((4096,), jnp.int32),
            pltpu.VMEM((WLCAP,), jnp.int32),
            pltpu.VMEM((WLCAP,), jnp.int32),
            pltpu.VMEM((HLCAP,), jnp.int32),
            pltpu.VMEM((HLCAP,), jnp.int32),
            pltpu.VMEM((STG_ROWS, 128), jnp.float32),
            pltpu.VMEM((2, 128), jnp.int32),
            pltpu.VMEM((4, 8, CW), jnp.float32),
            pltpu.VMEM((4, 8, CW), jnp.float32),
            pltpu.VMEM((64, 128), jnp.float32),
            pltpu.SemaphoreType.DMA,
            pltpu.SemaphoreType.DMA,
        ],
        compiler_params=cp,
    )
    ubuf, ibuf = k1(tu, ti, user, pos, negf, tailu, taili)
    k2 = pl.kernel(
        _bpr_dots_body,
        out_type=jax.ShapeDtypeStruct((1024, 128), jnp.float32),
        mesh=mesh,
        scratch_types=[
            pltpu.VMEM((128, 128), jnp.float32),
            pltpu.VMEM((128, 128), jnp.float32),
            pltpu.VMEM((512, 128), jnp.float32),
            pltpu.VMEM((8, 128), jnp.float32),
            pltpu.SemaphoreType.DMA,
        ],
        compiler_params=cp,
    )
    outp = k2(ubuf, ibuf)
    return outp.reshape(BATCH, 2 * NEG)


def kernel(user, pos_item, neg_item, user_table, item_table):
    user = user.astype(jnp.int32)
    pos = pos_item.astype(jnp.int32)
    negf = neg_item.astype(jnp.int32).reshape(BATCH * NEG)
    tu = user_table.T.reshape(4, 8, NUSER)
    ti = item_table.T.reshape(4, 8, NUSER)
    tailu = jnp.pad(user_table[SWEEP_END:], ((0, 0), (0, 128 - D)))
    taili = jnp.pad(item_table[SWEEP_END:], ((0, 0), (0, 128 - D)))
    return _bpr(user, pos, negf, tu, ti, tailu, taili)


# speedup vs baseline: 1.7468x; 1.1423x over previous
import functools

import jax
import jax.numpy as jnp
from jax import lax
from jax.experimental import pallas as pl
from jax.experimental.pallas import tpu as pltpu
from jax.experimental.pallas import tpu_sc as plsc

BATCH = 16384
D = 32
NEG = 4
NW = 32
NUSER = 1000000
RANGE = 31232
CW = 512
NCH = 62
SWEEP_END = 999936
WLCAP = 8192
HLCAP = 2048
STG_ROWS = 144
U_DUMMY = BATCH
I_DUMMY = 5 * BATCH


def _iota16():
    return lax.iota(jnp.int32, 16)


def _splat(v):
    return jnp.full((16,), v, jnp.int32)


def _scalar(v16):
    return lax.squeeze(lax.slice(v16, (0,), (1,)), (0,))


def _popc(mask):
    return _scalar(plsc.all_reduce_population_count(mask))


def _scan_src(src_hbm, nbatches, slot_off, lo_v, hi_v, idxb, wli, wls, wcnt):
    lane = _iota16()

    def batch_body(bi, wc):
        pltpu.sync_copy(src_hbm.at[pl.ds(bi * 4096, 4096)], idxb)

        def group_body(k, wc2):
            wi = idxb[pl.ds(k * 16, 16)]
            m = (wi >= lo_v) & (wi < hi_v)
            off = jnp.minimum(wc2, WLCAP - 1)
            plsc.store_compressed(wli.at[pl.ds(off, 16)], wi, mask=m)
            slot = _splat(slot_off) + bi * 4096 + k * 16 + lane
            plsc.store_compressed(wls.at[pl.ds(off, 16)], slot, mask=m)
            return wc2 + _popc(m)

        return lax.fori_loop(0, 256, group_body, wc)

    return lax.fori_loop(0, nbatches, batch_body, wcnt)


def _bpr_extract_body(tu, ti, user_h, pos_h, negf_h, tailu_h, taili_h,
                      ubuf, ibuf,
                      idxb, wli, wls, hlu, hls, stg, sl2, chk0, chk1, tailv,
                      semc, semf):
    c_ax = lax.axis_index("c")
    s_ax = lax.axis_index("s")
    wid = s_ax * 2 + c_ax
    lane = _iota16()
    lo = wid * RANGE
    lo_v = _splat(0) + lo

    def dummy_slots(row, dummy):
        for g in range(8):
            plsc.store_scatter(sl2, [_splat(row), g * 16 + lane],
                               _splat(dummy), mask=None)

    def fire(buf_hbm):
        cp = pltpu.async_copy(stg.at[pl.ds(0, 128)], buf_hbm.at[sl2.at[0]],
                              semf)
        cp.wait()

    def stage_b(chkb, tiled_chunk, hn, scnt0, buf_hbm, dummy):
        def g_body(g, scnt):
            ul = hlu[pl.ds(g * 16, 16)]
            sl = hls[pl.ds(g * 16, 16)]
            m = (g * 16 + lane) < hn
            p = jnp.minimum(scnt + lane, _splat(STG_ROWS - 1))
            for d in range(D):
                dv = _splat(d)
                if tiled_chunk:
                    v = plsc.load_gather(
                        chkb, [_splat(d >> 3), _splat(d & 7), ul], mask=m)
                else:
                    v = plsc.load_gather(chkb, [ul, dv], mask=m)
                plsc.store_scatter(stg, [p, dv], v, mask=m)
            plsc.store_scatter(sl2, [jnp.right_shift(p, 7),
                                     jnp.bitwise_and(p, _splat(127))],
                               sl, mask=m)
            scnt2 = scnt + _popc(m)
            fired = scnt2 >= 128

            @pl.when(fired)
            def _():
                fire(buf_hbm)
                for r in range(16):
                    for q in range(8):
                        stg[r, pl.ds(q * 16, 16)] = (
                            stg[128 + r, pl.ds(q * 16, 16)])
                for q in range(8):
                    sl2[0, pl.ds(q * 16, 16)] = sl2[1, pl.ds(q * 16, 16)]
                dummy_slots(1, dummy)

            return jnp.where(fired, scnt2 - 128, scnt2)

        return lax.fori_loop(0, (hn + 15) // 16, g_body, scnt0)

    def stage_a(clo_v, cw, wn):
        def k_body(k, hcnt):
            wi = wli[pl.ds(k * 16, 16)]
            m = ((wi >= clo_v) & (wi < clo_v + cw)
                 & ((k * 16 + lane) < wn))
            off = jnp.minimum(hcnt, HLCAP - 1)
            plsc.store_compressed(hlu.at[pl.ds(off, 16)], wi - clo_v, mask=m)
            ws = wls[pl.ds(k * 16, 16)]
            plsc.store_compressed(hls.at[pl.ds(off, 16)], ws, mask=m)
            return hcnt + _popc(m)

        return lax.fori_loop(0, (wn + 15) // 16, k_body, 0)

    def run_phase(table_v, tail_hbm, sources, buf_hbm, dummy):
        hi_v = jnp.where(wid == 31, NUSER, lo + RANGE) + _splat(0)
        wcnt = 0
        for (src, nb, soff) in sources:
            wcnt = _scan_src(src, nb, soff, lo_v, hi_v,
                             idxb, wli, wls, wcnt)
        dummy_slots(0, dummy)
        dummy_slots(1, dummy)
        pltpu.sync_copy(tail_hbm, tailv)

        def start_chunk(c, chkb):
            s = pl.multiple_of(lo + c * CW, 128)
            pltpu.async_copy(table_v.at[:, :, pl.ds(s, CW)], chkb, semc)

        def drain_chunk(c, chkb):
            s = pl.multiple_of(lo + c * CW, 128)
            pltpu.make_async_copy(table_v.at[:, :, pl.ds(s, CW)],
                                  chkb, semc).wait()

        start_chunk(0, chk0)
        start_chunk(1, chk1)

        def pair_body(c2, scnt):
            for b, chkb in ((0, chk0), (1, chk1)):
                c = 2 * c2 + b
                drain_chunk(c, chkb)
                hcnt = stage_a(_splat(0) + (lo + c * CW), CW, wcnt)
                scnt = stage_b(chkb, True, hcnt, scnt, buf_hbm, dummy)

                @pl.when(c2 < 30)
                def _():
                    start_chunk(c + 2, chkb)
            return scnt

        scnt = lax.fori_loop(0, NCH // 2, pair_body, 0)
        hcnt = stage_a(_splat(SWEEP_END), NUSER - SWEEP_END, wcnt)
        scnt = stage_b(tailv, False, hcnt, scnt, buf_hbm, dummy)
        fire(buf_hbm)

    run_phase(tu, tailu_h, [(user_h, 4, 0)], ubuf, U_DUMMY)
    run_phase(ti, taili_h, [(pos_h, 4, 0), (negf_h, 16, BATCH)], ibuf,
              I_DUMMY)


def _bpr_dots_body(ubuf, ibuf, outp, ulines, plines, nlines, outv, sem):
    c_ax = lax.axis_index("c")
    s_ax = lax.axis_index("s")
    wid = s_ax * 2 + c_ax
    lane = _iota16()
    base = wid * 512

    def sb_body(sb, carry):
        sbase = base + sb * 128
        pltpu.sync_copy(ubuf.at[pl.ds(sbase, 128)], ulines)
        pltpu.sync_copy(ibuf.at[pl.ds(sbase, 128)], plines)
        pltpu.sync_copy(ibuf.at[pl.ds(BATCH + sbase * 4, 512)], nlines)

        def g_body(g, carry2):
            lb = g * 16 + lane
            accp = jnp.zeros((16,), jnp.float32)
            accn = [jnp.zeros((16,), jnp.float32) for _ in range(NEG)]
            for d in range(D):
                dv = _splat(d)
                u = plsc.load_gather(ulines, [lb, dv])
                p = plsc.load_gather(plines, [lb, dv])
                accp = accp + u * p
                for j in range(NEG):
                    n = plsc.load_gather(nlines, [lb * NEG + j, dv])
                    accn[j] = accn[j] + u * n
            orow = jnp.right_shift(lb, 4)
            ocol0 = jnp.bitwise_and(lb, _splat(15)) * 8
            for cc in range(NEG):
                plsc.store_scatter(outv, [orow, ocol0 + cc], accp, mask=None)
            for j in range(NEG):
                plsc.store_scatter(outv, [orow, ocol0 + NEG + j], accn[j],
                                   mask=None)
            return carry2

        lax.fori_loop(0, 8, g_body, 0)
        pltpu.sync_copy(outv, outp.at[pl.ds(wid * 32 + sb * 8, 8)])
        return carry

    lax.fori_loop(0, 4, sb_body, 0)


@jax.jit
def _bpr(user, pos, negf, tu, ti, tailu, taili):
    mesh = plsc.VectorSubcoreMesh(core_axis_name="c", subcore_axis_name="s")
    cp = pltpu.CompilerParams(needs_layout_passes=False,
                              use_tc_tiling_on_sc=True)
    k1 = pl.kernel(
        _bpr_extract_body,
        out_type=(jax.ShapeDtypeStruct((BATCH + 1, 128), jnp.float32),
                  jax.ShapeDtypeStruct((5 * BATCH + 1, 128), jnp.float32)),
        mesh=mesh,
        scratch_types=[
            pltpu.VMEM((4096,), jnp.int32),
            pltpu.VMEM((WLCAP + 16,), jnp.int32),
            pltpu.VMEM((WLCAP + 16,), jnp.int32),
            pltpu.VMEM((HLCAP + 16,), jnp.int32),
            pltpu.VMEM((HLCAP + 16,), jnp.int32),
            pltpu.VMEM((STG_ROWS, 128), jnp.float32),
            pltpu.VMEM((2, 128), jnp.int32),
            pltpu.VMEM((4, 8, CW), jnp.float32),
            pltpu.VMEM((4, 8, CW), jnp.float32),
            pltpu.VMEM((64, 128), jnp.float32),
            pltpu.SemaphoreType.DMA,
            pltpu.SemaphoreType.DMA,
        ],
        compiler_params=cp,
    )
    ubuf, ibuf = k1(tu, ti, user, pos, negf, tailu, taili)
    k2 = pl.kernel(
        _bpr_dots_body,
        out_type=jax.ShapeDtypeStruct((1024, 128), jnp.float32),
        mesh=mesh,
        scratch_types=[
            pltpu.VMEM((128, 128), jnp.float32),
            pltpu.VMEM((128, 128), jnp.float32),
            pltpu.VMEM((512, 128), jnp.float32),
            pltpu.VMEM((8, 128), jnp.float32),
            pltpu.SemaphoreType.DMA,
        ],
        compiler_params=cp,
    )
    outp = k2(ubuf, ibuf)
    return outp.reshape(BATCH, 2 * NEG)


def kernel(user, pos_item, neg_item, user_table, item_table):
    user = user.astype(jnp.int32)
    pos = pos_item.astype(jnp.int32)
    negf = neg_item.astype(jnp.int32).reshape(BATCH * NEG)
    tu = user_table.T.reshape(4, 8, NUSER)
    ti = item_table.T.reshape(4, 8, NUSER)
    tailu = jnp.pad(user_table[SWEEP_END:], ((0, 0), (0, 128 - D)))
    taili = jnp.pad(item_table[SWEEP_END:], ((0, 0), (0, 128 - D)))
    return _bpr(user, pos, negf, tu, ti, tailu, taili)


# speedup vs baseline: 1.9541x; 1.1187x over previous
import functools

import jax
import jax.numpy as jnp
from jax import lax
from jax.experimental import pallas as pl
from jax.experimental.pallas import tpu as pltpu
from jax.experimental.pallas import tpu_sc as plsc

BATCH = 16384
D = 32
NEG = 4
NW = 32
NUSER = 1000000
RANGE = 31232
CW = 1024
NCH = 31
SWEEP_END = 999936
WLCAP = 8192
HLCAP = 2048
STG_ROWS = 144
U_DUMMY = BATCH
I_DUMMY = 5 * BATCH


def _iota16():
    return lax.iota(jnp.int32, 16)


def _splat(v):
    return jnp.full((16,), v, jnp.int32)


def _scalar(v16):
    return lax.squeeze(lax.slice(v16, (0,), (1,)), (0,))


def _popc(mask):
    return _scalar(plsc.all_reduce_population_count(mask))


def _scan_src(src_hbm, nbatches, slot_off, lo_v, hi_v, idxb, wli, wls, wcnt):
    lane = _iota16()

    def batch_body(bi, wc):
        pltpu.sync_copy(src_hbm.at[pl.ds(bi * 4096, 4096)], idxb)

        def group_body(k, wc2):
            wi = idxb[pl.ds(k * 16, 16)]
            m = (wi >= lo_v) & (wi < hi_v)
            off = jnp.minimum(wc2, WLCAP - 1)
            plsc.store_compressed(wli.at[pl.ds(off, 16)], wi, mask=m)
            slot = _splat(slot_off) + bi * 4096 + k * 16 + lane
            plsc.store_compressed(wls.at[pl.ds(off, 16)], slot, mask=m)
            return wc2 + _popc(m)

        return lax.fori_loop(0, 256, group_body, wc)

    return lax.fori_loop(0, nbatches, batch_body, wcnt)


def _bpr_extract_body(tu, ti, user_h, pos_h, negf_h, tailu_h, taili_h,
                      ubuf, ibuf,
                      idxb, wli, wls, hlu, hls, stg, sl2, chk0, chk1, tailv,
                      semc, semf):
    c_ax = lax.axis_index("c")
    s_ax = lax.axis_index("s")
    wid = s_ax * 2 + c_ax
    lane = _iota16()
    lo = wid * RANGE
    lo_v = _splat(0) + lo

    def dummy_slots(row, dummy):
        for g in range(8):
            plsc.store_scatter(sl2, [_splat(row), g * 16 + lane],
                               _splat(dummy), mask=None)

    def fire(buf_hbm):
        cp = pltpu.async_copy(stg.at[pl.ds(0, 128)], buf_hbm.at[sl2.at[0]],
                              semf)
        cp.wait()

    def stage_b(chkb, tiled_chunk, hn, scnt0, buf_hbm, dummy):
        def g_body(g, scnt):
            ul = hlu[pl.ds(g * 16, 16)]
            sl = hls[pl.ds(g * 16, 16)]
            m = (g * 16 + lane) < hn
            p = jnp.minimum(scnt + lane, _splat(STG_ROWS - 1))
            for d in range(D):
                dv = _splat(d)
                if tiled_chunk:
                    v = plsc.load_gather(
                        chkb, [_splat(d >> 3), _splat(d & 7), ul], mask=m)
                else:
                    v = plsc.load_gather(chkb, [ul, dv], mask=m)
                plsc.store_scatter(stg, [p, dv], v, mask=m)
            plsc.store_scatter(sl2, [jnp.right_shift(p, 7),
                                     jnp.bitwise_and(p, _splat(127))],
                               sl, mask=m)
            scnt2 = scnt + _popc(m)
            fired = scnt2 >= 128

            @pl.when(fired)
            def _():
                fire(buf_hbm)
                for r in range(16):
                    for q in range(8):
                        stg[r, pl.ds(q * 16, 16)] = (
                            stg[128 + r, pl.ds(q * 16, 16)])
                for q in range(8):
                    sl2[0, pl.ds(q * 16, 16)] = sl2[1, pl.ds(q * 16, 16)]
                dummy_slots(1, dummy)

            return jnp.where(fired, scnt2 - 128, scnt2)

        return lax.fori_loop(0, (hn + 15) // 16, g_body, scnt0)

    def stage_a(clo_v, cw, wn):
        def k_body(k, hcnt):
            wi = wli[pl.ds(k * 16, 16)]
            m = ((wi >= clo_v) & (wi < clo_v + cw)
                 & ((k * 16 + lane) < wn))
            off = jnp.minimum(hcnt, HLCAP - 1)
            plsc.store_compressed(hlu.at[pl.ds(off, 16)], wi - clo_v, mask=m)
            ws = wls[pl.ds(k * 16, 16)]
            plsc.store_compressed(hls.at[pl.ds(off, 16)], ws, mask=m)
            return hcnt + _popc(m)

        return lax.fori_loop(0, (wn + 15) // 16, k_body, 0)

    def run_phase(table_v, tail_hbm, sources, buf_hbm, dummy):
        hi_v = jnp.where(wid == 31, NUSER, lo + RANGE) + _splat(0)
        wcnt = 0
        for (src, nb, soff) in sources:
            wcnt = _scan_src(src, nb, soff, lo_v, hi_v,
                             idxb, wli, wls, wcnt)
        dummy_slots(0, dummy)
        dummy_slots(1, dummy)
        pltpu.sync_copy(tail_hbm, tailv)

        def start_chunk(c, chkb):
            s = pl.multiple_of(lo + c * CW, 128)
            pltpu.async_copy(table_v.at[:, :, pl.ds(s, CW)], chkb, semc)

        def drain_chunk(c, chkb):
            s = pl.multiple_of(lo + c * CW, 128)
            pltpu.make_async_copy(table_v.at[:, :, pl.ds(s, CW)],
                                  chkb, semc).wait()

        start_chunk(0, chk0)
        start_chunk(1, chk1)

        def pair_body(c2, scnt):
            for b, chkb in ((0, chk0), (1, chk1)):
                c = 2 * c2 + b
                drain_chunk(c, chkb)
                hcnt = stage_a(_splat(0) + (lo + c * CW), CW, wcnt)
                scnt = stage_b(chkb, True, hcnt, scnt, buf_hbm, dummy)
                if b == 0:
                    start_chunk(c + 2, chkb)
                else:
                    @pl.when(c2 < NCH // 2 - 1)
                    def _():
                        start_chunk(c + 2, chkb)
            return scnt

        scnt = lax.fori_loop(0, NCH // 2, pair_body, 0)
        drain_chunk(NCH - 1, chk0)
        hcnt = stage_a(_splat(0) + (lo + (NCH - 1) * CW), CW, wcnt)
        scnt = stage_b(chk0, True, hcnt, scnt, buf_hbm, dummy)
        hcnt = stage_a(_splat(SWEEP_END), NUSER - SWEEP_END, wcnt)
        scnt = stage_b(tailv, False, hcnt, scnt, buf_hbm, dummy)
        fire(buf_hbm)

    run_phase(tu, tailu_h, [(user_h, 4, 0)], ubuf, U_DUMMY)
    run_phase(ti, taili_h, [(pos_h, 4, 0), (negf_h, 16, BATCH)], ibuf,
              I_DUMMY)


def _bpr_dots_body(ubuf, ibuf, outp, ulines, plines, nlines, outv, sem):
    c_ax = lax.axis_index("c")
    s_ax = lax.axis_index("s")
    wid = s_ax * 2 + c_ax
    lane = _iota16()
    base = wid * 512

    def sb_body(sb, carry):
        sbase = base + sb * 128
        pltpu.sync_copy(ubuf.at[pl.ds(sbase, 128)], ulines)
        pltpu.sync_copy(ibuf.at[pl.ds(sbase, 128)], plines)
        pltpu.sync_copy(ibuf.at[pl.ds(BATCH + sbase * 4, 512)], nlines)

        def g_body(g, carry2):
            lb = g * 16 + lane
            accp = jnp.zeros((16,), jnp.float32)
            accn = [jnp.zeros((16,), jnp.float32) for _ in range(NEG)]
            for d in range(D):
                dv = _splat(d)
                u = plsc.load_gather(ulines, [lb, dv])
                p = plsc.load_gather(plines, [lb, dv])
                accp = accp + u * p
                for j in range(NEG):
                    n = plsc.load_gather(nlines, [lb * NEG + j, dv])
                    accn[j] = accn[j] + u * n
            orow = jnp.right_shift(lb, 4)
            ocol0 = jnp.bitwise_and(lb, _splat(15)) * 8
            for cc in range(NEG):
                plsc.store_scatter(outv, [orow, ocol0 + cc], accp, mask=None)
            for j in range(NEG):
                plsc.store_scatter(outv, [orow, ocol0 + NEG + j], accn[j],
                                   mask=None)
            return carry2

        lax.fori_loop(0, 8, g_body, 0)
        pltpu.sync_copy(outv, outp.at[pl.ds(wid * 32 + sb * 8, 8)])
        return carry

    lax.fori_loop(0, 4, sb_body, 0)


@jax.jit
def _bpr(user, pos, negf, tu, ti, tailu, taili):
    mesh = plsc.VectorSubcoreMesh(core_axis_name="c", subcore_axis_name="s")
    cp = pltpu.CompilerParams(needs_layout_passes=False,
                              use_tc_tiling_on_sc=True)
    k1 = pl.kernel(
        _bpr_extract_body,
        out_type=(jax.ShapeDtypeStruct((BATCH + 1, 128), jnp.float32),
                  jax.ShapeDtypeStruct((5 * BATCH + 1, 128), jnp.float32)),
        mesh=mesh,
        scratch_types=[
            pltpu.VMEM((4096,), jnp.int32),
            pltpu.VMEM((WLCAP + 16,), jnp.int32),
            pltpu.VMEM((WLCAP + 16,), jnp.int32),
            pltpu.VMEM((HLCAP + 16,), jnp.int32),
            pltpu.VMEM((HLCAP + 16,), jnp.int32),
            pltpu.VMEM((STG_ROWS, 128), jnp.float32),
            pltpu.VMEM((2, 128), jnp.int32),
            pltpu.VMEM((4, 8, CW), jnp.float32),
            pltpu.VMEM((4, 8, CW), jnp.float32),
            pltpu.VMEM((64, 128), jnp.float32),
            pltpu.SemaphoreType.DMA,
            pltpu.SemaphoreType.DMA,
        ],
        compiler_params=cp,
    )
    ubuf, ibuf = k1(tu, ti, user, pos, negf, tailu, taili)
    k2 = pl.kernel(
        _bpr_dots_body,
        out_type=jax.ShapeDtypeStruct((1024, 128), jnp.float32),
        mesh=mesh,
        scratch_types=[
            pltpu.VMEM((128, 128), jnp.float32),
            pltpu.VMEM((128, 128), jnp.float32),
            pltpu.VMEM((512, 128), jnp.float32),
            pltpu.VMEM((8, 128), jnp.float32),
            pltpu.SemaphoreType.DMA,
        ],
        compiler_params=cp,
    )
    outp = k2(ubuf, ibuf)
    return outp.reshape(BATCH, 2 * NEG)


def kernel(user, pos_item, neg_item, user_table, item_table):
    user = user.astype(jnp.int32)
    pos = pos_item.astype(jnp.int32)
    negf = neg_item.astype(jnp.int32).reshape(BATCH * NEG)
    tu = user_table.T.reshape(4, 8, NUSER)
    ti = item_table.T.reshape(4, 8, NUSER)
    tailu = jnp.pad(user_table[SWEEP_END:], ((0, 0), (0, 128 - D)))
    taili = jnp.pad(item_table[SWEEP_END:], ((0, 0), (0, 128 - D)))
    return _bpr(user, pos, negf, tu, ti, tailu, taili)
